# baseline (speedup 1.0000x reference)
import functools

import jax
import jax.numpy as jnp
from jax import lax
from jax.experimental import pallas as pl
from jax.experimental.pallas import tpu as pltpu
from jax.experimental.pallas import tpu_sc as plsc

_NC = 2
_NS = 16
_NW = _NC * _NS
_LANES = 16
_CHUNK = 128


def _scores_kernel(B, D, users_hbm, items_hbm, ut_hbm, it_hbm, out_hbm,
                   uraw, iraw, ubuf, ibuf, acc, idx_sem,
                   usem0, usem1, isem0, isem1):
    bpw = B // _NW
    nchunk = bpw // _CHUNK
    nvec = bpw // _LANES
    wid = lax.axis_index("s") * _NC + lax.axis_index("c")
    base = wid * bpw
    usems = (usem0, usem1)
    isems = (isem0, isem1)

    idx_copies = []
    for j in range(nchunk):
        idx_copies.append(pltpu.async_copy(
            users_hbm.at[pl.ds(base + j * _CHUNK, _CHUNK)], uraw.at[j],
            idx_sem))
        idx_copies.append(pltpu.async_copy(
            items_hbm.at[pl.ds(base + j * _CHUNK, _CHUNK)], iraw.at[j],
            idx_sem))
    for c in idx_copies:
        c.wait()

    def fire(d):
        s = d % 2
        for j in range(nchunk):
            dsl = pl.ds(j * _CHUNK, _CHUNK)
            pltpu.async_copy(
                ut_hbm.at[d].at[uraw.at[j]], ubuf.at[s].at[dsl], usems[s])
            pltpu.async_copy(
                it_hbm.at[d].at[iraw.at[j]], ibuf.at[s].at[dsl], isems[s])

    def drain(d):
        s = d % 2
        pltpu.make_async_copy(
            ut_hbm.at[d].at[pl.ds(0, bpw)], ubuf.at[s], usems[s]).wait()
        pltpu.make_async_copy(
            it_hbm.at[d].at[pl.ds(0, bpw)], ibuf.at[s], isems[s]).wait()

    fire(0)
    fire(1)

    for d in range(D):
        s = d % 2
        drain(d)

        if d == 0:
            def slab0(v, _):
                sl = pl.ds(v * _LANES, _LANES)
                acc[sl] = ubuf[s, sl] * ibuf[s, sl]
                return 0
            lax.fori_loop(0, nvec, slab0, 0)
        else:
            def slab(v, _):
                sl = pl.ds(v * _LANES, _LANES)
                acc[sl] = acc[sl] + ubuf[s, sl] * ibuf[s, sl]
                return 0
            lax.fori_loop(0, nvec, slab, 0)

        if d + 2 < D:
            fire(d + 2)

    def sig(v, _):
        sl = pl.ds(v * _LANES, _LANES)
        acc[sl] = 1.0 / (1.0 + jnp.exp(-acc[sl]))
        return 0
    lax.fori_loop(0, nvec, sig, 0)

    pltpu.sync_copy(acc, out_hbm.at[pl.ds(base, bpw)])


def kernel(users, items, user_table, item_table):
    B = users.shape[0]
    V, D = user_table.shape
    bpw = B // _NW
    nchunk = bpw // _CHUNK
    ut_t = user_table.T
    it_t = item_table.T
    mesh = plsc.VectorSubcoreMesh(core_axis_name="c", subcore_axis_name="s")

    run = functools.partial(
        pl.kernel,
        mesh=mesh,
        compiler_params=pltpu.CompilerParams(
            needs_layout_passes=False, use_tc_tiling_on_sc=False),
        out_type=jax.ShapeDtypeStruct((B,), jnp.float32),
        scratch_types=[
            pltpu.VMEM((nchunk, _CHUNK), jnp.int32),
            pltpu.VMEM((nchunk, _CHUNK), jnp.int32),
            pltpu.VMEM((2, bpw), jnp.float32),
            pltpu.VMEM((2, bpw), jnp.float32),
            pltpu.VMEM((bpw,), jnp.float32),
            pltpu.SemaphoreType.DMA,
            pltpu.SemaphoreType.DMA,
            pltpu.SemaphoreType.DMA,
            pltpu.SemaphoreType.DMA,
            pltpu.SemaphoreType.DMA,
        ],
    )(functools.partial(_scores_kernel, B, D))
    return run(users, items, ut_t, it_t)


# speedup vs baseline: 21.6497x; 21.6497x over previous
import functools

import jax
import jax.numpy as jnp
from jax import lax
from jax.experimental import pallas as pl
from jax.experimental.pallas import tpu as pltpu
from jax.experimental.pallas import tpu_sc as plsc

_NC = 2
_NS = 16
_NW = _NC * _NS
_LANES = 16
_G = 4


def _scores_kernel(B, D, users_hbm, items_hbm, ut_hbm, it_hbm, out_hbm,
                   uvm, ivm, ubuf, ibuf, odots, oscr,
                   idx_sem, usem0, usem1, isem0, isem1):
    bpw = B // _NW
    ngroup = bpw // _G
    wid = lax.axis_index("s") * _NC + lax.axis_index("c")
    base = wid * bpw
    usems = (usem0, usem1)
    isems = (isem0, isem1)

    cu = pltpu.async_copy(users_hbm.at[pl.ds(base, bpw)],
                          uvm.at[pl.ds(0, bpw)], idx_sem)
    ci = pltpu.async_copy(items_hbm.at[pl.ds(base, bpw)],
                          ivm.at[pl.ds(0, bpw)], idx_sem)
    cu.wait()
    ci.wait()

    dlanes0 = lax.iota(jnp.int32, _LANES)
    dlanes1 = dlanes0 + _LANES
    lane_mask = dlanes0 == 0

    def fire_one(u, i, j, s):
        ub = pl.multiple_of((u >> 7) * 128, 128)
        ib = pl.multiple_of((i >> 7) * 128, 128)
        pltpu.async_copy(ut_hbm.at[:, pl.ds(ub, 128)], ubuf.at[s, j],
                         usems[s])
        pltpu.async_copy(it_hbm.at[:, pl.ds(ib, 128)], ibuf.at[s, j],
                         isems[s])

    def drain(s):
        pltpu.make_async_copy(ut_hbm.at[:, pl.ds(0, _G * 128)], ubuf.at[s],
                              usems[s]).wait()
        pltpu.make_async_copy(it_hbm.at[:, pl.ds(0, _G * 128)], ibuf.at[s],
                              isems[s]).wait()

    def process_one(u, i, j, s, pos):
        s_spl = jnp.full((_LANES,), s, jnp.int32)
        j_spl = jnp.full((_LANES,), j, jnp.int32)
        urin = jnp.full((_LANES,), u & 127, jnp.int32)
        irin = jnp.full((_LANES,), i & 127, jnp.int32)
        u0 = plsc.load_gather(ubuf, [s_spl, j_spl, dlanes0, urin])
        u1 = plsc.load_gather(ubuf, [s_spl, j_spl, dlanes1, urin])
        i0 = plsc.load_gather(ibuf, [s_spl, j_spl, dlanes0, irin])
        i1 = plsc.load_gather(ibuf, [s_spl, j_spl, dlanes1, irin])
        p = u0 * i0 + u1 * i1
        dot = jnp.sum(p)
        plsc.store_scatter(odots, [jnp.full((_LANES,), pos, jnp.int32)],
                           jnp.full((_LANES,), dot, jnp.float32),
                           mask=lane_mask)

    vec_u0 = uvm[pl.ds(0, _LANES)]
    vec_i0 = ivm[pl.ds(0, _LANES)]
    for j in range(_G):
        fire_one(vec_u0[j], vec_i0[j], j, 0)
    for j in range(_G):
        fire_one(vec_u0[_G + j], vec_i0[_G + j], j, 1)

    def pair(k, _):
        e0 = k * 2 * _G
        uvec = uvm[pl.ds(e0, _LANES)]
        ivec = ivm[pl.ds(e0, _LANES)]

        drain(0)
        for j in range(_G):
            process_one(uvec[j], ivec[j], j, 0, e0 + j)

        @pl.when(e0 + 2 * _G < bpw)
        def _():
            for j in range(_G):
                fire_one(uvec[2 * _G + j], ivec[2 * _G + j], j, 0)

        drain(1)
        for j in range(_G):
            process_one(uvec[_G + j], ivec[_G + j], j, 1, e0 + _G + j)

        @pl.when(e0 + 3 * _G < bpw)
        def _():
            for j in range(_G):
                fire_one(uvec[3 * _G + j], ivec[3 * _G + j], j, 1)

        return 0

    lax.fori_loop(0, ngroup // 2, pair, 0)

    def sig(v, _):
        sl = pl.ds(v * _LANES, _LANES)
        oscr[sl] = 1.0 / (1.0 + jnp.exp(-odots[sl]))
        return 0

    lax.fori_loop(0, bpw // _LANES, sig, 0)

    pltpu.sync_copy(oscr, out_hbm.at[pl.ds(base, bpw)])


def kernel(users, items, user_table, item_table):
    B = users.shape[0]
    V, D = user_table.shape
    bpw = B // _NW
    ut_t = user_table.T
    it_t = item_table.T
    mesh = plsc.VectorSubcoreMesh(core_axis_name="c", subcore_axis_name="s")

    run = functools.partial(
        pl.kernel,
        mesh=mesh,
        compiler_params=pltpu.CompilerParams(
            needs_layout_passes=False, use_tc_tiling_on_sc=True),
        out_type=jax.ShapeDtypeStruct((B,), jnp.float32),
        scratch_types=[
            pltpu.VMEM((bpw + _LANES,), jnp.int32),
            pltpu.VMEM((bpw + _LANES,), jnp.int32),
            pltpu.VMEM((2, _G, D, 128), jnp.float32),
            pltpu.VMEM((2, _G, D, 128), jnp.float32),
            pltpu.VMEM((bpw,), jnp.float32),
            pltpu.VMEM((bpw,), jnp.float32),
            pltpu.SemaphoreType.DMA,
            pltpu.SemaphoreType.DMA,
            pltpu.SemaphoreType.DMA,
            pltpu.SemaphoreType.DMA,
            pltpu.SemaphoreType.DMA,
        ],
    )(functools.partial(_scores_kernel, B, D))
    return run(users, items, ut_t, it_t)


# speedup vs baseline: 23.9272x; 1.1052x over previous
import functools

import jax
import jax.numpy as jnp
from jax import lax
from jax.experimental import pallas as pl
from jax.experimental.pallas import tpu as pltpu
from jax.experimental.pallas import tpu_sc as plsc

_NC = 2
_NS = 16
_NW = _NC * _NS
_LANES = 16
_G = 4
_S = 3


def _scores_kernel(B, D, users_hbm, items_hbm, ut_hbm, it_hbm, out_hbm,
                   uvm, ivm, ubuf, ibuf, odots, oscr,
                   idx_sem, usem0, usem1, usem2, isem0, isem1, isem2):
    bpw = B // _NW
    ngroup = bpw // _G
    wid = lax.axis_index("s") * _NC + lax.axis_index("c")
    base = wid * bpw
    usems = (usem0, usem1, usem2)
    isems = (isem0, isem1, isem2)

    cu = pltpu.async_copy(users_hbm.at[pl.ds(base, bpw)],
                          uvm.at[pl.ds(0, bpw)], idx_sem)
    ci = pltpu.async_copy(items_hbm.at[pl.ds(base, bpw)],
                          ivm.at[pl.ds(0, bpw)], idx_sem)
    cu.wait()
    ci.wait()

    dlanes0 = lax.iota(jnp.int32, _LANES)
    dlanes1 = dlanes0 + _LANES
    lane_mask = dlanes0 == 0

    def fire(g, s):
        uvec = uvm[pl.ds(g * _G, _LANES)]
        ivec = ivm[pl.ds(g * _G, _LANES)]
        for j in range(_G):
            u = uvec[j]
            i = ivec[j]
            ub = pl.multiple_of((u >> 7) * 128, 128)
            ib = pl.multiple_of((i >> 7) * 128, 128)
            pltpu.async_copy(ut_hbm.at[:, pl.ds(ub, 128)], ubuf.at[s, j],
                             usems[s])
            pltpu.async_copy(it_hbm.at[:, pl.ds(ib, 128)], ibuf.at[s, j],
                             isems[s])

    def drain(s):
        pltpu.make_async_copy(ut_hbm.at[:, pl.ds(0, _G * 128)], ubuf.at[s],
                              usems[s]).wait()
        pltpu.make_async_copy(it_hbm.at[:, pl.ds(0, _G * 128)], ibuf.at[s],
                              isems[s]).wait()

    def process(g, s):
        uvec = uvm[pl.ds(g * _G, _LANES)]
        ivec = ivm[pl.ds(g * _G, _LANES)]
        s_spl = jnp.full((_LANES,), s, jnp.int32)
        for j in range(_G):
            u = uvec[j]
            i = ivec[j]
            j_spl = jnp.full((_LANES,), j, jnp.int32)
            urin = jnp.full((_LANES,), u & 127, jnp.int32)
            irin = jnp.full((_LANES,), i & 127, jnp.int32)
            u0 = plsc.load_gather(ubuf, [s_spl, j_spl, dlanes0, urin])
            u1 = plsc.load_gather(ubuf, [s_spl, j_spl, dlanes1, urin])
            i0 = plsc.load_gather(ibuf, [s_spl, j_spl, dlanes0, irin])
            i1 = plsc.load_gather(ibuf, [s_spl, j_spl, dlanes1, irin])
            p = u0 * i0 + u1 * i1
            dot = jnp.sum(p)
            plsc.store_scatter(
                odots, [jnp.full((_LANES,), g * _G + j, jnp.int32)],
                jnp.full((_LANES,), dot, jnp.float32), mask=lane_mask)

    for s in range(_S):
        fire(s, s)

    def triple(k, _):
        for m in range(_S):
            g = k * _S + m
            drain(m)
            process(g, m)

            @pl.when(g + _S < ngroup)
            def _():
                fire(g + _S, m)
        return 0

    lax.fori_loop(0, ngroup // _S, triple, 0)
    for g in range(ngroup - ngroup % _S, ngroup):
        drain(g % _S)
        process(g, g % _S)

    def sig(v, _):
        sl = pl.ds(v * _LANES, _LANES)
        oscr[sl] = 1.0 / (1.0 + jnp.exp(-odots[sl]))
        return 0

    lax.fori_loop(0, bpw // _LANES, sig, 0)

    pltpu.sync_copy(oscr, out_hbm.at[pl.ds(base, bpw)])


def kernel(users, items, user_table, item_table):
    B = users.shape[0]
    V, D = user_table.shape
    bpw = B // _NW
    ut_t = user_table.T
    it_t = item_table.T
    mesh = plsc.VectorSubcoreMesh(core_axis_name="c", subcore_axis_name="s")

    run = functools.partial(
        pl.kernel,
        mesh=mesh,
        compiler_params=pltpu.CompilerParams(
            needs_layout_passes=False, use_tc_tiling_on_sc=True),
        out_type=jax.ShapeDtypeStruct((B,), jnp.float32),
        scratch_types=[
            pltpu.VMEM((bpw + _LANES,), jnp.int32),
            pltpu.VMEM((bpw + _LANES,), jnp.int32),
            pltpu.VMEM((_S, _G, D, 128), jnp.float32),
            pltpu.VMEM((_S, _G, D, 128), jnp.float32),
            pltpu.VMEM((bpw,), jnp.float32),
            pltpu.VMEM((bpw,), jnp.float32),
            pltpu.SemaphoreType.DMA,
            pltpu.SemaphoreType.DMA,
            pltpu.SemaphoreType.DMA,
            pltpu.SemaphoreType.DMA,
            pltpu.SemaphoreType.DMA,
            pltpu.SemaphoreType.DMA,
            pltpu.SemaphoreType.DMA,
        ],
    )(functools.partial(_scores_kernel, B, D))
    return run(users, items, ut_t, it_t)
